# baseline (speedup 1.0000x reference)
import jax
import jax.numpy as jnp
from jax.experimental import pallas as pl


def _conv2d(x, w):
    return jax.lax.conv_general_dilated(x, w, (1, 1), 'SAME', dimension_numbers=('NCHW', 'OIHW', 'NCHW'))


def _bnorm(h, eps=1e-5):
    m = jnp.mean(h, axis=(0, 2, 3), keepdims=True)
    v = jnp.var(h, axis=(0, 2, 3), keepdims=True)
    return (h - m) / jnp.sqrt(v + eps)


def _res_block(f, W1, W2):
    h = jax.nn.relu(_bnorm(_conv2d(f, W1)))
    h = jax.nn.relu(_bnorm(_conv2d(h, W2)))
    return jax.nn.relu(f + h)


def _segment_softmax(w, seg, num):
    m = jax.ops.segment_max(w, seg, num_segments=num)
    e = jnp.exp(w - m[seg])
    s = jax.ops.segment_sum(e, seg, num_segments=num)
    return e / (s[seg] + 1e-9)


def _softmax_head(seg_map):
    def body(x_ref, o_ref):
        x = x_ref[...]
        m = jnp.max(x, axis=-1, keepdims=True)
        e = jnp.exp(x - m)
        s = jnp.sum(e, axis=-1, keepdims=True)
        o_ref[...] = (e / s)[:, :8]

    return pl.pallas_call(
        body,
        out_shape=jax.ShapeDtypeStruct((seg_map.shape[0], 8), jnp.float32),
    )(seg_map)


def kernel(x, edges, init_state, W_bb1, b_bb1, W_bb2, b_bb2, W_k1, W_k2, W_kp, b_kp, W_q1, W_q2, W_qp):
    B, H, Wd, C = x.shape
    xc = jnp.transpose(x, (0, 3, 1, 2))
    f = jax.nn.relu(_conv2d(xc, W_bb1) + b_bb1[None, :, None, None])
    f = jax.nn.relu(_conv2d(f, W_bb2) + b_bb2[None, :, None, None])
    kf = _conv2d(_res_block(f, W_k1, W_k2), W_kp) + b_kp[None, :, None, None]
    qf = _conv2d(_res_block(f, W_q1, W_q2), W_qp)
    D = kf.shape[1]
    Nt = B * H * Wd
    ksf = jnp.transpose(kf.reshape(B, D, H * Wd), (0, 2, 1)).reshape(Nt, D)
    qsf = jnp.transpose(qf.reshape(B, D, H * Wd), (0, 2, 1)).reshape(Nt, D)
    e0 = edges[0]
    e1 = edges[1]
    qe = qsf[e0]
    ke = ksf[e1]
    sim = jnp.sum(qe * ke, axis=-1) / (jnp.linalg.norm(qe, axis=-1) * jnp.linalg.norm(ke, axis=-1) + 1e-8)
    w_edge = _segment_softmax(sim, e0, Nt)
    state = init_state
    for _ in range(32):
        msg = state[e1] * w_edge[:, None]
        agg = jax.ops.segment_sum(msg, e0, num_segments=Nt)
        state = agg / (jnp.linalg.norm(agg, axis=-1, keepdims=True) + 1e-8)
    masks = _softmax_head(state).reshape(B, H, Wd, 8)
    return jnp.transpose(masks, (0, 3, 1, 2))


# speedup vs baseline: 5.3850x; 5.3850x over previous
import functools

import jax
import jax.numpy as jnp
from jax import lax
from jax.experimental import pallas as pl
from jax.experimental.pallas import tpu as pltpu
from jax.experimental.pallas import tpu_sc as plsc

NC, NS, L = 2, 16, 16
NW = NC * NS
NT = 8192
Q = 128
D = 64
E = 204800
EPT = E // NW
CH = 128
NCHUNK = EPT // CH
ROWS = NT // NW
STRIPE = NT // NS

_MESH = plsc.VectorSubcoreMesh(core_axis_name="c", subcore_axis_name="s")


def _lanes():
    return lax.iota(jnp.int32, L)


def _take(v, idx):
    return lax.gather(
        v, idx[:, None],
        lax.GatherDimensionNumbers(offset_dims=(), collapsed_slice_dims=(0,),
                                   start_index_map=(0,)),
        slice_sizes=(1,), mode=lax.GatherScatterMode.PROMISE_IN_BOUNDS)


def _bcast_lane(v, l):
    return _take(v, _lanes() * 0 + l)


def _lanesum(v):
    for s in (1, 2, 4, 8):
        v = v + _take(v, _lanes() ^ s)
    return v


def _vrsqrt(x):
    i = lax.bitcast_convert_type(x, jnp.int32)
    y = lax.bitcast_convert_type(jnp.int32(0x5F3759DF) - (i >> 1), jnp.float32)
    for _ in range(3):
        y = y * (1.5 - 0.5 * x * y * y)
    return y


def _wid():
    return lax.axis_index("s") * NC + lax.axis_index("c")


def _a1_body(qsf_ref, ksf_ref, e0_ref, e1_ref,
             p_out, den_out,
             e0_v, e1_v, q_v, k_v, p_v, st_v, den_sh, sem):
    c = lax.axis_index("c")
    s = lax.axis_index("s")
    t = s * NC + c
    zero = jnp.zeros((L,), jnp.float32)

    def _z(i, cc):
        st_v[pl.ds(i * L, L)] = zero
        return cc
    lax.fori_loop(0, STRIPE // L, _z, 0)
    pltpu.sync_copy(st_v, den_sh.at[pl.ds(s * STRIPE, STRIPE)])
    plsc.subcore_barrier()

    def _chunk(ch, cc):
        off = t * EPT + ch * CH
        pltpu.sync_copy(e0_ref.at[pl.ds(off, CH)], e0_v)
        pltpu.sync_copy(e1_ref.at[pl.ds(off, CH)], e1_v)
        pltpu.async_copy(qsf_ref.at[e0_v], q_v, sem).wait()
        pltpu.async_copy(ksf_ref.at[e1_v], k_v, sem).wait()

        def _grp(g, c2):
            pacc = zero
            for l in range(L):
                e = g * L + l
                dot = zero
                nq = zero
                nk = zero
                for kk in range(D // L):
                    qv = q_v[e, pl.ds(kk * L, L)]
                    kv = k_v[e, pl.ds(kk * L, L)]
                    dot = dot + qv * kv
                    nq = nq + qv * qv
                    nk = nk + kv * kv
                dot = _lanesum(dot)
                nq = _lanesum(nq)
                nk = _lanesum(nk)
                prod = nq * nk
                sq = prod * _vrsqrt(prod)
                sim = dot / (sq + 1e-8)
                pe = jnp.exp(sim)
                pacc = jnp.where(_lanes() == l, pe, pacc)
            p_v[pl.ds(g * L, L)] = pacc
            return c2

        lax.fori_loop(0, CH // L, _grp, 0)
        pltpu.sync_copy(p_v, p_out.at[pl.ds(off, CH)])
        pltpu.async_copy(p_v, den_sh.at[e0_v], sem, add=True).wait()
        return cc

    lax.fori_loop(0, NCHUNK, _chunk, 0)
    plsc.subcore_barrier()
    pltpu.sync_copy(den_sh.at[pl.ds(s * STRIPE, STRIPE)], st_v)
    pltpu.sync_copy(st_v, den_out.at[pl.ds(c * NT + s * STRIPE, STRIPE)])


_a1 = functools.partial(
    pl.kernel,
    out_type=(jax.ShapeDtypeStruct((E,), jnp.float32),
              jax.ShapeDtypeStruct((NC * NT,), jnp.float32)),
    mesh=_MESH,
    scratch_types=[
        pltpu.VMEM((CH,), jnp.int32),
        pltpu.VMEM((CH,), jnp.int32),
        pltpu.VMEM((CH, Q), jnp.float32),
        pltpu.VMEM((CH, Q), jnp.float32),
        pltpu.VMEM((CH,), jnp.float32),
        pltpu.VMEM((STRIPE,), jnp.float32),
        pltpu.VMEM_SHARED((NT,), jnp.float32),
        pltpu.SemaphoreType.DMA,
    ],
)(_a1_body)


def _a2_body(den_ref, inv_out, a_v, b_v):
    b = _wid()
    pltpu.sync_copy(den_ref.at[pl.ds(b * ROWS, ROWS)], a_v)
    pltpu.sync_copy(den_ref.at[pl.ds(NT + b * ROWS, ROWS)], b_v)

    def _r(i, cc):
        v = a_v[pl.ds(i * L, L)] + b_v[pl.ds(i * L, L)]
        a_v[pl.ds(i * L, L)] = 1.0 / (v + 1e-9)
        return cc
    lax.fori_loop(0, ROWS // L, _r, 0)
    pltpu.sync_copy(a_v, inv_out.at[pl.ds(b * ROWS, ROWS)])


_a2 = functools.partial(
    pl.kernel,
    out_type=jax.ShapeDtypeStruct((NT,), jnp.float32),
    mesh=_MESH,
    scratch_types=[
        pltpu.VMEM((ROWS,), jnp.float32),
        pltpu.VMEM((ROWS,), jnp.float32),
    ],
)(_a2_body)


def _a3_body(p_ref, e0_ref, inv_ref, w_out, iv_v, p_v, e0_v, sem):
    t = _wid()

    def _chunk(ch, cc):
        off = t * EPT + ch * CH
        pltpu.sync_copy(p_ref.at[pl.ds(off, CH)], p_v)
        pltpu.sync_copy(e0_ref.at[pl.ds(off, CH)], e0_v)
        pltpu.async_copy(inv_ref.at[e0_v], iv_v, sem).wait()

        def _grp(g, c2):
            p_v[pl.ds(g * L, L)] = (
                p_v[pl.ds(g * L, L)] * iv_v[pl.ds(g * L, L)])
            return c2
        lax.fori_loop(0, CH // L, _grp, 0)
        pltpu.sync_copy(p_v, w_out.at[pl.ds(off, CH)])
        return cc

    lax.fori_loop(0, NCHUNK, _chunk, 0)


_a3 = functools.partial(
    pl.kernel,
    out_type=jax.ShapeDtypeStruct((E,), jnp.float32),
    mesh=_MESH,
    scratch_types=[
        pltpu.VMEM((CH,), jnp.float32),
        pltpu.VMEM((CH,), jnp.float32),
        pltpu.VMEM((CH,), jnp.int32),
        pltpu.SemaphoreType.DMA,
    ],
)(_a3_body)


def _b1_body(state_ref, e0_ref, e1_ref, w_ref,
             part_out,
             e0_v, e1_v, w_v, rows_v, zz_v, agg_sh, sem):
    c = lax.axis_index("c")
    s = lax.axis_index("s")
    t = s * NC + c
    zero = jnp.zeros((L,), jnp.float32)

    def _z(i, cc):
        for k in range(Q // L):
            zz_v[i, pl.ds(k * L, L)] = zero
        return cc
    lax.fori_loop(0, CH, _z, 0)
    for j in range(STRIPE // CH):
        pltpu.sync_copy(zz_v, agg_sh.at[pl.ds(s * STRIPE + j * CH, CH)])
    plsc.subcore_barrier()

    def _chunk(ch, cc):
        off = t * EPT + ch * CH
        pltpu.sync_copy(e0_ref.at[pl.ds(off, CH)], e0_v)
        pltpu.sync_copy(e1_ref.at[pl.ds(off, CH)], e1_v)
        pltpu.sync_copy(w_ref.at[pl.ds(off, CH)], w_v)
        pltpu.async_copy(state_ref.at[e1_v], rows_v, sem).wait()

        def _grp(g, c2):
            wv = w_v[pl.ds(g * L, L)]
            for l in range(L):
                wl = _bcast_lane(wv, l)
                e = g * L + l
                for k in range(Q // L):
                    rows_v[e, pl.ds(k * L, L)] = (
                        rows_v[e, pl.ds(k * L, L)] * wl)
            return c2
        lax.fori_loop(0, CH // L, _grp, 0)
        pltpu.async_copy(rows_v, agg_sh.at[e0_v], sem, add=True).wait()
        return cc

    lax.fori_loop(0, NCHUNK, _chunk, 0)
    plsc.subcore_barrier()
    for j in range(STRIPE // CH):
        pltpu.sync_copy(agg_sh.at[pl.ds(s * STRIPE + j * CH, CH)], zz_v)
        pltpu.sync_copy(
            zz_v, part_out.at[pl.ds(c * NT + s * STRIPE + j * CH, CH)])


_b1 = functools.partial(
    pl.kernel,
    out_type=jax.ShapeDtypeStruct((NC * NT, Q), jnp.float32),
    mesh=_MESH,
    scratch_types=[
        pltpu.VMEM((CH,), jnp.int32),
        pltpu.VMEM((CH,), jnp.int32),
        pltpu.VMEM((CH,), jnp.float32),
        pltpu.VMEM((CH, Q), jnp.float32),
        pltpu.VMEM((CH, Q), jnp.float32),
        pltpu.VMEM_SHARED((NT, Q), jnp.float32),
        pltpu.SemaphoreType.DMA,
    ],
)(_b1_body)


def _b2_body(part_ref, out_ref, pa_v, pb_v):
    b = _wid()
    pltpu.sync_copy(part_ref.at[pl.ds(b * ROWS, ROWS)], pa_v)
    pltpu.sync_copy(part_ref.at[pl.ds(NT + b * ROWS, ROWS)], pb_v)

    def _r(r, cc):
        vs = []
        ss = jnp.zeros((L,), jnp.float32)
        for k in range(Q // L):
            v = pa_v[r, pl.ds(k * L, L)] + pb_v[r, pl.ds(k * L, L)]
            vs.append(v)
            ss = ss + v * v
        ss = _lanesum(ss)
        nf = 1.0 / (ss * _vrsqrt(ss) + 1e-8)
        for k in range(Q // L):
            pa_v[r, pl.ds(k * L, L)] = vs[k] * nf
        return cc
    lax.fori_loop(0, ROWS, _r, 0)
    pltpu.sync_copy(pa_v, out_ref.at[pl.ds(b * ROWS, ROWS)])


_b2 = functools.partial(
    pl.kernel,
    out_type=jax.ShapeDtypeStruct((NT, Q), jnp.float32),
    mesh=_MESH,
    scratch_types=[
        pltpu.VMEM((ROWS, Q), jnp.float32),
        pltpu.VMEM((ROWS, Q), jnp.float32),
    ],
)(_b2_body)


def _conv2d(x, w):
    return jax.lax.conv_general_dilated(
        x, w, (1, 1), 'SAME', dimension_numbers=('NCHW', 'OIHW', 'NCHW'))


def _bnorm(h, eps=1e-5):
    m = jnp.mean(h, axis=(0, 2, 3), keepdims=True)
    v = jnp.var(h, axis=(0, 2, 3), keepdims=True)
    return (h - m) / jnp.sqrt(v + eps)


def _res_block(f, W1, W2):
    h = jax.nn.relu(_bnorm(_conv2d(f, W1)))
    h = jax.nn.relu(_bnorm(_conv2d(h, W2)))
    return jax.nn.relu(f + h)


def _softmax_head(seg_map):
    def body(x_ref, o_ref):
        x = x_ref[...]
        m = jnp.max(x, axis=-1, keepdims=True)
        e = jnp.exp(x - m)
        su = jnp.sum(e, axis=-1, keepdims=True)
        o_ref[...] = (e / su)[:, :8]

    return pl.pallas_call(
        body,
        out_shape=jax.ShapeDtypeStruct((seg_map.shape[0], 8), jnp.float32),
    )(seg_map)


def kernel(x, edges, init_state, W_bb1, b_bb1, W_bb2, b_bb2,
           W_k1, W_k2, W_kp, b_kp, W_q1, W_q2, W_qp):
    B, H, Wd, C = x.shape
    xc = jnp.transpose(x, (0, 3, 1, 2))
    f = jax.nn.relu(_conv2d(xc, W_bb1) + b_bb1[None, :, None, None])
    f = jax.nn.relu(_conv2d(f, W_bb2) + b_bb2[None, :, None, None])
    kf = _conv2d(_res_block(f, W_k1, W_k2), W_kp) + b_kp[None, :, None, None]
    qf = _conv2d(_res_block(f, W_q1, W_q2), W_qp)
    ksf = jnp.transpose(kf.reshape(B, D, H * Wd), (0, 2, 1)).reshape(NT, D)
    qsf = jnp.transpose(qf.reshape(B, D, H * Wd), (0, 2, 1)).reshape(NT, D)
    e0 = edges[0].astype(jnp.int32)
    e1 = edges[1].astype(jnp.int32)

    qsf_p = jnp.pad(qsf, ((0, 0), (0, Q - D)))
    ksf_p = jnp.pad(ksf, ((0, 0), (0, Q - D)))
    p, den = _a1(qsf_p, ksf_p, e0, e1)
    inv = _a2(den)
    w = _a3(p, e0, inv)

    state = init_state
    for _ in range(32):
        part = _b1(state, e0, e1, w)
        state = _b2(part)

    masks = _softmax_head(state).reshape(B, H, Wd, 8)
    return jnp.transpose(masks, (0, 3, 1, 2))
